# Initial kernel scaffold; baseline (speedup 1.0000x reference)
import jax
import jax.numpy as jnp
from jax.experimental import pallas as pl


def kernel(src_sequences, target_sequences, src_table, tgt_table):
    raise NotImplementedError("write your pallas kernel here")


# speedup vs baseline: 4.4656x; 4.4656x over previous
import functools

import jax
import jax.numpy as jnp
from jax import lax
from jax.experimental import pallas as pl
from jax.experimental.pallas import tpu as pltpu
from jax.experimental.pallas import tpu_sc as plsc

BATCH = 4
SEQ_LEN = 8192
D_MODEL = 768

NUM_CORES = 2
NUM_SUBCORES = 16
NUM_WORKERS = NUM_CORES * NUM_SUBCORES
ROWS_PER_WORKER = SEQ_LEN // NUM_WORKERS
CHUNK = 128
CHUNKS_PER_WORKER = ROWS_PER_WORKER // CHUNK


def _pe_broadcast_kernel(src_table_hbm, tgt_table_hbm, src_out_hbm,
                         tgt_out_hbm, buf):
    wid = lax.axis_index("s") * NUM_CORES + lax.axis_index("c")
    base = wid * ROWS_PER_WORKER
    for table_hbm, out_hbm in ((src_table_hbm, src_out_hbm),
                               (tgt_table_hbm, tgt_out_hbm)):
        for c in range(CHUNKS_PER_WORKER):
            start = base + c * CHUNK
            pltpu.sync_copy(table_hbm.at[pl.ds(start, CHUNK)], buf)
            for b in range(BATCH):
                pltpu.sync_copy(buf, out_hbm.at[b, pl.ds(start, CHUNK)])


@functools.partial(
    pl.kernel,
    out_type=(
        jax.ShapeDtypeStruct((BATCH, SEQ_LEN, D_MODEL), jnp.float32),
        jax.ShapeDtypeStruct((BATCH, SEQ_LEN, D_MODEL), jnp.float32),
    ),
    mesh=plsc.VectorSubcoreMesh(core_axis_name="c", subcore_axis_name="s"),
    scratch_types=[pltpu.VMEM((CHUNK, D_MODEL), jnp.float32)],
)
def _pe_broadcast(src_table_hbm, tgt_table_hbm, src_out_hbm, tgt_out_hbm,
                  buf):
    _pe_broadcast_kernel(src_table_hbm, tgt_table_hbm, src_out_hbm,
                         tgt_out_hbm, buf)


def kernel(src_sequences, target_sequences, src_table, tgt_table):
    del src_sequences, target_sequences
    return _pe_broadcast(src_table, tgt_table)
